# baseline (speedup 1.0000x reference)
import jax
import jax.numpy as jnp
from jax import lax
from jax.experimental import pallas as pl
from jax.experimental.pallas import tpu as pltpu
from jax.experimental.pallas import tpu_sc as plsc

N = 10000
NPAD = 10240
HID = 128
T_IN = 30
TPAD = 32
G4 = 4 * HID
OUT_CH = 21
OUTP = 128
TIME_STEPS = 7
N_QUANTILES = 3
E = 320000
CHUNK = 128
NCHUNK = 79
EPT = NCHUNK * CHUNK
EPAD = 32 * EPT
ROWS_PER_TILE = NPAD // 16
BOM = 10
KB_PT = NPAD // 32
KB_CH = 8
AK0, AK1 = 57, 101
KB0, KB1 = 20, 60
BLK = 1024


def _lstm_body(xt_ref, wih_ref, whh_ref, b_ref, out_ref):
    blk = out_ref.shape[0]

    def step(t, carry):
        h, c = carry
        xt = xt_ref[t, :][:, None]
        gates = (xt * wih_ref[...]
                 + jnp.dot(h.astype(jnp.bfloat16), whh_ref[...],
                           preferred_element_type=jnp.float32)
                 + b_ref[...])
        i = jax.nn.sigmoid(gates[:, 0 * HID:1 * HID])
        f = jax.nn.sigmoid(gates[:, 1 * HID:2 * HID])
        g = jnp.tanh(gates[:, 2 * HID:3 * HID])
        o = jax.nn.sigmoid(gates[:, 3 * HID:4 * HID])
        c = f * c + i * g
        h = o * jnp.tanh(c)
        return (h, c)

    h0 = jnp.zeros((blk, HID), jnp.float32)
    c0 = jnp.zeros((blk, HID), jnp.float32)
    h, _ = lax.fori_loop(0, T_IN, step, (h0, c0))
    out_ref[...] = h


def _lstm_call(x_t, wih_row, whhT, bias):
    return pl.pallas_call(
        _lstm_body,
        grid=(NPAD // BLK,),
        in_specs=[
            pl.BlockSpec((TPAD, BLK), lambda i: (0, i)),
            pl.BlockSpec((1, G4), lambda i: (0, 0)),
            pl.BlockSpec((HID, G4), lambda i: (0, 0)),
            pl.BlockSpec((1, G4), lambda i: (0, 0)),
        ],
        out_specs=pl.BlockSpec((BLK, HID), lambda i: (i, 0)),
        out_shape=jax.ShapeDtypeStruct((NPAD, HID), jnp.float32),
    )(x_t, wih_row, whhT, bias)


def _agg_body(feat, srcp, dstp, zf, part,
              sidx0, sidx1, didx0, didx1, rows0, rows1,
              semi0, semi1, semg0, semg1, acc_sh):
    cid = lax.axis_index("c")
    sid = lax.axis_index("s")
    wid = cid * 16 + sid
    r0 = pl.multiple_of(sid * ROWS_PER_TILE, ROWS_PER_TILE)
    for k in range(ROWS_PER_TILE // CHUNK):
        rk = pl.multiple_of(r0 + k * CHUNK, CHUNK)
        pltpu.sync_copy(zf.at[pl.ds(rk, CHUNK)], rows0)
        pltpu.sync_copy(rows0, acc_sh.at[pl.ds(rk, CHUNK)])
    plsc.subcore_barrier()

    nchunk = jnp.where(cid == 0, AK0, AK1)
    base = pl.multiple_of(
        jnp.where(cid == 0, sid * AK0, 16 * AK0 + sid * AK1) * CHUNK, CHUNK)

    def idx_start(c, sb, db, semi):
        off = pl.multiple_of(base + c * CHUNK, CHUNK)
        pltpu.async_copy(srcp.at[pl.ds(off, CHUNK)], sb, semi)
        pltpu.async_copy(dstp.at[pl.ds(off, CHUNK)], db, semi)

    def idx_wait(c, sb, db, semi):
        off = pl.multiple_of(base + c * CHUNK, CHUNK)
        pltpu.make_async_copy(srcp.at[pl.ds(off, CHUNK)], sb, semi).wait()
        pltpu.make_async_copy(dstp.at[pl.ds(off, CHUNK)], db, semi).wait()

    idx_start(0, sidx0, didx0, semi0)
    idx_start(1, sidx1, didx1, semi1)
    idx_wait(0, sidx0, didx0, semi0)
    pltpu.async_copy(feat.at[sidx0], rows0, semg0)
    idx_wait(1, sidx1, didx1, semi1)

    def pair(i, carry):
        a = 2 * i
        pltpu.async_copy(feat.at[sidx1], rows1, semg1)
        pltpu.make_async_copy(feat.at[sidx0], rows0, semg0).wait()
        pltpu.sync_copy(rows0, acc_sh.at[didx0], add=True)
        idx_start(a + 2, sidx0, didx0, semi0)
        pltpu.make_async_copy(feat.at[sidx1], rows1, semg1).wait()
        pltpu.sync_copy(rows1, acc_sh.at[didx1], add=True)

        @pl.when(a + 3 < nchunk)
        def _():
            idx_start(a + 3, sidx1, didx1, semi1)

        idx_wait(a + 2, sidx0, didx0, semi0)

        @pl.when(a + 3 < nchunk)
        def _():
            idx_wait(a + 3, sidx1, didx1, semi1)

        pltpu.async_copy(feat.at[sidx0], rows0, semg0)
        return carry

    lax.fori_loop(0, (nchunk - 1) // 2, pair, 0)
    pltpu.make_async_copy(feat.at[sidx0], rows0, semg0).wait()
    pltpu.sync_copy(rows0, acc_sh.at[didx0], add=True)
    plsc.subcore_barrier()
    for k in range(ROWS_PER_TILE // CHUNK):
        rk = pl.multiple_of(r0 + k * CHUNK, CHUNK)
        pltpu.sync_copy(acc_sh.at[pl.ds(rk, CHUNK)], rows0)
        pltpu.sync_copy(rows0, part.at[cid, pl.ds(rk, CHUNK)])


def _agg_call(feat, srcp, dstp, zf):
    return pl.kernel(
        _agg_body,
        out_type=jax.ShapeDtypeStruct((2, NPAD, HID), jnp.float32),
        mesh=plsc.VectorSubcoreMesh(core_axis_name="c", subcore_axis_name="s",
                                    num_cores=2, num_subcores=16),
        scratch_types=[
            pltpu.VMEM((CHUNK,), jnp.int32),
            pltpu.VMEM((CHUNK,), jnp.int32),
            pltpu.VMEM((CHUNK,), jnp.int32),
            pltpu.VMEM((CHUNK,), jnp.int32),
            pltpu.VMEM((CHUNK, HID), jnp.float32),
            pltpu.VMEM((CHUNK, HID), jnp.float32),
            pltpu.SemaphoreType.DMA,
            pltpu.SemaphoreType.DMA,
            pltpu.SemaphoreType.DMA,
            pltpu.SemaphoreType.DMA,
            pltpu.VMEM_SHARED((NPAD, HID), jnp.float32),
        ],
    )(feat, srcp, dstp, zf)


def _cnt_body(dstp, zf, ones_h, cntp,
              didx0, didx1, ones_v, rows_v, semi0, semi1, acc_sh):
    cid = lax.axis_index("c")
    sid = lax.axis_index("s")
    wid = cid * 16 + sid
    r0 = pl.multiple_of(sid * ROWS_PER_TILE, ROWS_PER_TILE)
    for k in range(ROWS_PER_TILE // CHUNK):
        rk = pl.multiple_of(r0 + k * CHUNK, CHUNK)
        pltpu.sync_copy(zf.at[pl.ds(rk, CHUNK)], rows_v)
        pltpu.sync_copy(rows_v, acc_sh.at[pl.ds(rk, CHUNK)])
    pltpu.sync_copy(ones_h, ones_v)
    base = wid * EPT

    def idx_start(c, db, semi):
        off = pl.multiple_of(base + c * CHUNK, CHUNK)
        pltpu.async_copy(dstp.at[pl.ds(off, CHUNK)], db, semi)

    def idx_wait(c, db, semi):
        off = pl.multiple_of(base + c * CHUNK, CHUNK)
        pltpu.make_async_copy(dstp.at[pl.ds(off, CHUNK)], db, semi).wait()

    idx_start(0, didx0, semi0)
    idx_start(1, didx1, semi1)
    plsc.subcore_barrier()

    def pair(i, carry):
        a = 2 * i
        idx_wait(a, didx0, semi0)
        pltpu.sync_copy(ones_v, acc_sh.at[didx0], add=True)
        idx_start(a + 2, didx0, semi0)
        idx_wait(a + 1, didx1, semi1)
        pltpu.sync_copy(ones_v, acc_sh.at[didx1], add=True)

        @pl.when(a + 3 < NCHUNK)
        def _():
            idx_start(a + 3, didx1, semi1)

        return carry

    lax.fori_loop(0, (NCHUNK - 1) // 2, pair, 0)
    idx_wait(NCHUNK - 1, didx0, semi0)
    pltpu.sync_copy(ones_v, acc_sh.at[didx0], add=True)
    plsc.subcore_barrier()
    for k in range(ROWS_PER_TILE // CHUNK):
        rk = pl.multiple_of(r0 + k * CHUNK, CHUNK)
        pltpu.sync_copy(acc_sh.at[pl.ds(rk, CHUNK)], rows_v)
        pltpu.sync_copy(rows_v, cntp.at[cid, pl.ds(rk, CHUNK)])


def _cnt_call(dstp, zf, ones_h):
    return pl.kernel(
        _cnt_body,
        out_type=jax.ShapeDtypeStruct((2, NPAD, HID), jnp.float32),
        mesh=plsc.VectorSubcoreMesh(core_axis_name="c", subcore_axis_name="s",
                                    num_cores=2, num_subcores=16),
        scratch_types=[
            pltpu.VMEM((CHUNK,), jnp.int32),
            pltpu.VMEM((CHUNK,), jnp.int32),
            pltpu.VMEM((CHUNK, HID), jnp.float32),
            pltpu.VMEM((CHUNK, HID), jnp.float32),
            pltpu.SemaphoreType.DMA,
            pltpu.SemaphoreType.DMA,
            pltpu.VMEM_SHARED((NPAD, HID), jnp.float32),
        ],
    )(dstp, zf, ones_h)


def _sage1_body(p0, p1, c0, c1, x, wl, bl, wr, lnw, lnb, out):
    agg = (p0[...] + p1[...]) / jnp.maximum(c0[...] + c1[...], 1.0)
    y = (jnp.dot(agg, wl[...], preferred_element_type=jnp.float32)
         + bl[...]
         + jnp.dot(x[...], wr[...], preferred_element_type=jnp.float32))
    h1 = jnp.maximum(y, 0.0)
    mu = jnp.mean(h1, axis=1, keepdims=True)
    var = jnp.mean((h1 - mu) * (h1 - mu), axis=1, keepdims=True)
    x1 = (h1 - mu) * lax.rsqrt(var + 1e-5) * lnw[...] + lnb[...]
    out[...] = jnp.maximum(x1 + x[...], 0.0)


def _sage1_call(p0, p1, c0, c1, x, wlT, bl, wrT, lnw, lnb):
    spec_f = pl.BlockSpec((BLK, HID), lambda i: (i, 0))
    spec_w = pl.BlockSpec((HID, HID), lambda i: (0, 0))
    spec_b = pl.BlockSpec((1, HID), lambda i: (0, 0))
    return pl.pallas_call(
        _sage1_body,
        grid=(NPAD // BLK,),
        in_specs=[spec_f, spec_f, spec_f, spec_f, spec_f, spec_w, spec_b,
                  spec_w, spec_b, spec_b],
        out_specs=spec_f,
        out_shape=jax.ShapeDtypeStruct((NPAD, HID), jnp.float32),
    )(p0, p1, c0, c1, x, wlT, bl, wrT, lnw, lnb)


def _sage2_body(p0, p1, c0, c1, x1, res, wl, bl, wr, pw, pb, out):
    agg = (p0[...] + p1[...]) / jnp.maximum(c0[...] + c1[...], 1.0)
    y = (jnp.dot(agg, wl[...], preferred_element_type=jnp.float32)
         + bl[...]
         + jnp.dot(x1[...], wr[...], preferred_element_type=jnp.float32))
    xo = jnp.maximum(y + res[...], 0.0)
    o = jnp.dot(xo, pw[...], preferred_element_type=jnp.float32) + pb[...]
    rowid = (pl.program_id(0) * BLK
             + lax.broadcasted_iota(jnp.int32, (BLK, OUTP), 0))
    out[...] = jnp.where(rowid < N, o, 0.0)


def _sage2_call(p0, p1, c0, c1, x1, res, wl2T, bl2, wr2T, projWT, projb):
    spec_f = pl.BlockSpec((BLK, HID), lambda i: (i, 0))
    spec_w = pl.BlockSpec((HID, HID), lambda i: (0, 0))
    spec_b = pl.BlockSpec((1, HID), lambda i: (0, 0))
    return pl.pallas_call(
        _sage2_body,
        grid=(NPAD // BLK,),
        in_specs=[spec_f, spec_f, spec_f, spec_f, spec_f, spec_f, spec_w,
                  spec_b, spec_w,
                  pl.BlockSpec((HID, OUTP), lambda i: (0, 0)),
                  pl.BlockSpec((1, OUTP), lambda i: (0, 0))],
        out_specs=pl.BlockSpec((BLK, OUTP), lambda i: (i, 0)),
        out_shape=jax.ShapeDtypeStruct((NPAD, OUTP), jnp.float32),
    )(p0, p1, c0, c1, x1, res, wl2T, bl2, wr2T, projWT, projb)


KB_NCH = KB_PT // KB_CH


def _kb_sum(rows, out_v):
    for t in range(KB_CH):
        for half in range(2):
            acc = rows[t * BOM, pl.ds(half * 16, 16)]
            for j in range(1, BOM):
                acc = acc + rows[t * BOM + j, pl.ds(half * 16, 16)]
            out_v[t, pl.ds(half * 16, 16)] = acc


def _kb_body(proj, kbf, out, kidx, rows0, rows1, out_v, sem0, sem1):
    cid = lax.axis_index("c")
    sid = lax.axis_index("s")
    nch = jnp.where(cid == 0, KB0, KB1)
    tbase = pl.multiple_of(
        jnp.where(cid == 0, sid * KB0, 16 * KB0 + sid * KB1) * KB_CH, KB_CH)
    ibase = pl.multiple_of(tbase * BOM, KB_CH * BOM)
    pltpu.sync_copy(kbf.at[pl.ds(ibase, KB1 * KB_CH * BOM)], kidx)

    def emit(c, rows):
        _kb_sum(rows, out_v)
        toff = pl.multiple_of(tbase + c * KB_CH, KB_CH)
        pltpu.sync_copy(out_v, out.at[pl.ds(toff, KB_CH)])

    def gidx(c):
        return kidx.at[pl.ds(c * KB_CH * BOM, KB_CH * BOM)]

    pltpu.async_copy(proj.at[gidx(0)], rows0, sem0)

    def pair(i, carry):
        a = 2 * i
        pltpu.async_copy(proj.at[gidx(a + 1)], rows1, sem1)
        pltpu.make_async_copy(proj.at[gidx(a)], rows0, sem0).wait()
        emit(a, rows0)

        @pl.when(a + 2 < nch)
        def _():
            pltpu.async_copy(proj.at[gidx(a + 2)], rows0, sem0)

        pltpu.make_async_copy(proj.at[gidx(a + 1)], rows1, sem1).wait()
        emit(a + 1, rows1)
        return carry

    lax.fori_loop(0, nch // 2, pair, 0)


def _kb_call(proj, kbf):
    return pl.kernel(
        _kb_body,
        out_type=jax.ShapeDtypeStruct((NPAD, OUTP), jnp.float32),
        mesh=plsc.VectorSubcoreMesh(core_axis_name="c", subcore_axis_name="s",
                                    num_cores=2, num_subcores=16),
        scratch_types=[
            pltpu.VMEM((KB1 * KB_CH * BOM,), jnp.int32),
            pltpu.VMEM((KB_CH * BOM, OUTP), jnp.float32),
            pltpu.VMEM((KB_CH * BOM, OUTP), jnp.float32),
            pltpu.VMEM((KB_CH, OUTP), jnp.float32),
            pltpu.SemaphoreType.DMA,
            pltpu.SemaphoreType.DMA,
        ],
    )(proj, kbf)


def kernel(x_key, keybom, key_aggregation_status, edge_index,
           lstm_Wih, lstm_Whh, lstm_bih, lstm_bhh,
           sage1_Wl, sage1_bl, sage1_Wr, ln1_w, ln1_b,
           sage2_Wl, sage2_bl, sage2_Wr, proj_W, proj_b):
    f32 = jnp.float32
    x_t = jnp.zeros((TPAD, NPAD), f32).at[:T_IN, :N].set(x_key.T)
    wih_row = lstm_Wih.reshape(1, G4)
    whhT = lstm_Whh.T.astype(jnp.bfloat16)
    bias = (lstm_bih + lstm_bhh).reshape(1, G4)

    src = edge_index[0]
    dst = edge_index[1]
    srcp = jnp.full((EPAD,), N, jnp.int32).at[:E].set(src)
    dstp = jnp.full((EPAD,), N, jnp.int32).at[:E].set(dst)
    zf = jnp.zeros((NPAD, HID), f32)
    ones_h = jnp.ones((CHUNK, HID), f32)

    h = _lstm_call(x_t, wih_row, whhT, bias)
    res = h
    cnt = _cnt_call(dstp, zf, ones_h)

    part1 = _agg_call(h, srcp, dstp, zf)
    x1 = _sage1_call(part1[0], part1[1], cnt[0], cnt[1], h,
                     sage1_Wl.T, sage1_bl.reshape(1, HID), sage1_Wr.T,
                     ln1_w.reshape(1, HID), ln1_b.reshape(1, HID))

    part2 = _agg_call(x1, srcp, dstp, zf)
    projWT = jnp.zeros((HID, OUTP), f32).at[:, :OUT_CH].set(proj_W.T)
    projb = jnp.zeros((1, OUTP), f32).at[0, :OUT_CH].set(proj_b)
    po = _sage2_call(part2[0], part2[1], cnt[0], cnt[1], x1, res,
                     sage2_Wl.T, sage2_bl.reshape(1, HID), sage2_Wr.T,
                     projWT, projb)

    kb = jnp.where(keybom < 0, N, keybom)
    kbf = jnp.zeros((NPAD * BOM,), jnp.int32).at[:N * BOM].set(kb.reshape(-1))
    ko = _kb_call(po, kbf)
    return ko[:N, :OUT_CH].reshape(N, TIME_STEPS, N_QUANTILES)


# speedup vs baseline: 1.1144x; 1.1144x over previous
import jax
import jax.numpy as jnp
from jax import lax
from jax.experimental import pallas as pl
from jax.experimental.pallas import tpu as pltpu
from jax.experimental.pallas import tpu_sc as plsc

N = 10000
NPAD = 10240
HID = 128
T_IN = 30
TPAD = 32
G4 = 4 * HID
OUT_CH = 21
OUTP = 128
TIME_STEPS = 7
N_QUANTILES = 3
E = 320000
CHUNK = 128
NCHUNK = 79
EPT = NCHUNK * CHUNK
EPAD = 32 * EPT
ROWS_PER_TILE = NPAD // 16
BOM = 10
KB_PT = NPAD // 32
KB_CH = 8
AK0, AK1 = 101, 57
KB0, KB1 = 60, 20
KBMX = max(KB0, KB1)
KBF_LEN = NPAD * BOM + KBMX * KB_CH * BOM
BLK = 1024


def _lstm_body(xt_ref, wih_ref, whh_ref, b_ref, out_ref):
    blk = out_ref.shape[0]

    def step(t, carry):
        h, c = carry
        xt = xt_ref[t, :][:, None]
        gates = (xt * wih_ref[...]
                 + jnp.dot(h.astype(jnp.bfloat16), whh_ref[...],
                           preferred_element_type=jnp.float32)
                 + b_ref[...])
        i = jax.nn.sigmoid(gates[:, 0 * HID:1 * HID])
        f = jax.nn.sigmoid(gates[:, 1 * HID:2 * HID])
        g = jnp.tanh(gates[:, 2 * HID:3 * HID])
        o = jax.nn.sigmoid(gates[:, 3 * HID:4 * HID])
        c = f * c + i * g
        h = o * jnp.tanh(c)
        return (h, c)

    h0 = jnp.zeros((blk, HID), jnp.float32)
    c0 = jnp.zeros((blk, HID), jnp.float32)
    h, _ = lax.fori_loop(0, T_IN, step, (h0, c0))
    out_ref[...] = h


def _lstm_call(x_t, wih_row, whhT, bias):
    return pl.pallas_call(
        _lstm_body,
        grid=(NPAD // BLK,),
        in_specs=[
            pl.BlockSpec((TPAD, BLK), lambda i: (0, i)),
            pl.BlockSpec((1, G4), lambda i: (0, 0)),
            pl.BlockSpec((HID, G4), lambda i: (0, 0)),
            pl.BlockSpec((1, G4), lambda i: (0, 0)),
        ],
        out_specs=pl.BlockSpec((BLK, HID), lambda i: (i, 0)),
        out_shape=jax.ShapeDtypeStruct((NPAD, HID), jnp.float32),
    )(x_t, wih_row, whhT, bias)


def _agg_body(feat, srcp, dstp, zf, part,
              sidx0, sidx1, didx0, didx1, rows0, rows1,
              semi0, semi1, semg0, semg1, acc_sh):
    cid = lax.axis_index("c")
    sid = lax.axis_index("s")
    wid = cid * 16 + sid
    r0 = pl.multiple_of(sid * ROWS_PER_TILE, ROWS_PER_TILE)
    for k in range(ROWS_PER_TILE // CHUNK):
        rk = pl.multiple_of(r0 + k * CHUNK, CHUNK)
        pltpu.sync_copy(zf.at[pl.ds(rk, CHUNK)], rows0)
        pltpu.sync_copy(rows0, acc_sh.at[pl.ds(rk, CHUNK)])
    plsc.subcore_barrier()

    nchunk = jnp.where(cid == 0, AK0, AK1)
    base = pl.multiple_of(
        jnp.where(cid == 0, sid * AK0, 16 * AK0 + sid * AK1) * CHUNK, CHUNK)

    def idx_start(c, sb, db, semi):
        off = pl.multiple_of(base + c * CHUNK, CHUNK)
        pltpu.async_copy(srcp.at[pl.ds(off, CHUNK)], sb, semi)
        pltpu.async_copy(dstp.at[pl.ds(off, CHUNK)], db, semi)

    def idx_wait(c, sb, db, semi):
        off = pl.multiple_of(base + c * CHUNK, CHUNK)
        pltpu.make_async_copy(srcp.at[pl.ds(off, CHUNK)], sb, semi).wait()
        pltpu.make_async_copy(dstp.at[pl.ds(off, CHUNK)], db, semi).wait()

    idx_start(0, sidx0, didx0, semi0)
    idx_start(1, sidx1, didx1, semi1)
    idx_wait(0, sidx0, didx0, semi0)
    pltpu.async_copy(feat.at[sidx0], rows0, semg0)
    idx_wait(1, sidx1, didx1, semi1)

    def pair(i, carry):
        a = 2 * i
        pltpu.async_copy(feat.at[sidx1], rows1, semg1)
        pltpu.make_async_copy(feat.at[sidx0], rows0, semg0).wait()
        pltpu.sync_copy(rows0, acc_sh.at[didx0], add=True)
        idx_start(a + 2, sidx0, didx0, semi0)
        pltpu.make_async_copy(feat.at[sidx1], rows1, semg1).wait()
        pltpu.sync_copy(rows1, acc_sh.at[didx1], add=True)

        @pl.when(a + 3 < nchunk)
        def _():
            idx_start(a + 3, sidx1, didx1, semi1)

        idx_wait(a + 2, sidx0, didx0, semi0)

        @pl.when(a + 3 < nchunk)
        def _():
            idx_wait(a + 3, sidx1, didx1, semi1)

        pltpu.async_copy(feat.at[sidx0], rows0, semg0)
        return carry

    lax.fori_loop(0, (nchunk - 1) // 2, pair, 0)
    pltpu.make_async_copy(feat.at[sidx0], rows0, semg0).wait()
    pltpu.sync_copy(rows0, acc_sh.at[didx0], add=True)
    plsc.subcore_barrier()
    for k in range(ROWS_PER_TILE // CHUNK):
        rk = pl.multiple_of(r0 + k * CHUNK, CHUNK)
        pltpu.sync_copy(acc_sh.at[pl.ds(rk, CHUNK)], rows0)
        pltpu.sync_copy(rows0, part.at[cid, pl.ds(rk, CHUNK)])


def _agg_call(feat, srcp, dstp, zf):
    return pl.kernel(
        _agg_body,
        out_type=jax.ShapeDtypeStruct((2, NPAD, HID), jnp.float32),
        mesh=plsc.VectorSubcoreMesh(core_axis_name="c", subcore_axis_name="s",
                                    num_cores=2, num_subcores=16),
        scratch_types=[
            pltpu.VMEM((CHUNK,), jnp.int32),
            pltpu.VMEM((CHUNK,), jnp.int32),
            pltpu.VMEM((CHUNK,), jnp.int32),
            pltpu.VMEM((CHUNK,), jnp.int32),
            pltpu.VMEM((CHUNK, HID), jnp.float32),
            pltpu.VMEM((CHUNK, HID), jnp.float32),
            pltpu.SemaphoreType.DMA,
            pltpu.SemaphoreType.DMA,
            pltpu.SemaphoreType.DMA,
            pltpu.SemaphoreType.DMA,
            pltpu.VMEM_SHARED((NPAD, HID), jnp.float32),
        ],
    )(feat, srcp, dstp, zf)


def _cnt_body(dstp, zf, ones_h, cntp,
              didx0, didx1, ones_v, rows_v, semi0, semi1, acc_sh):
    cid = lax.axis_index("c")
    sid = lax.axis_index("s")
    wid = cid * 16 + sid
    r0 = pl.multiple_of(sid * ROWS_PER_TILE, ROWS_PER_TILE)
    for k in range(ROWS_PER_TILE // CHUNK):
        rk = pl.multiple_of(r0 + k * CHUNK, CHUNK)
        pltpu.sync_copy(zf.at[pl.ds(rk, CHUNK)], rows_v)
        pltpu.sync_copy(rows_v, acc_sh.at[pl.ds(rk, CHUNK)])
    pltpu.sync_copy(ones_h, ones_v)
    base = wid * EPT

    def idx_start(c, db, semi):
        off = pl.multiple_of(base + c * CHUNK, CHUNK)
        pltpu.async_copy(dstp.at[pl.ds(off, CHUNK)], db, semi)

    def idx_wait(c, db, semi):
        off = pl.multiple_of(base + c * CHUNK, CHUNK)
        pltpu.make_async_copy(dstp.at[pl.ds(off, CHUNK)], db, semi).wait()

    idx_start(0, didx0, semi0)
    idx_start(1, didx1, semi1)
    plsc.subcore_barrier()

    def pair(i, carry):
        a = 2 * i
        idx_wait(a, didx0, semi0)
        pltpu.sync_copy(ones_v, acc_sh.at[didx0], add=True)
        idx_start(a + 2, didx0, semi0)
        idx_wait(a + 1, didx1, semi1)
        pltpu.sync_copy(ones_v, acc_sh.at[didx1], add=True)

        @pl.when(a + 3 < NCHUNK)
        def _():
            idx_start(a + 3, didx1, semi1)

        return carry

    lax.fori_loop(0, (NCHUNK - 1) // 2, pair, 0)
    idx_wait(NCHUNK - 1, didx0, semi0)
    pltpu.sync_copy(ones_v, acc_sh.at[didx0], add=True)
    plsc.subcore_barrier()
    for k in range(ROWS_PER_TILE // CHUNK):
        rk = pl.multiple_of(r0 + k * CHUNK, CHUNK)
        pltpu.sync_copy(acc_sh.at[pl.ds(rk, CHUNK)], rows_v)
        pltpu.sync_copy(rows_v, cntp.at[cid, pl.ds(rk, CHUNK)])


def _cnt_call(dstp, zf, ones_h):
    return pl.kernel(
        _cnt_body,
        out_type=jax.ShapeDtypeStruct((2, NPAD, HID), jnp.float32),
        mesh=plsc.VectorSubcoreMesh(core_axis_name="c", subcore_axis_name="s",
                                    num_cores=2, num_subcores=16),
        scratch_types=[
            pltpu.VMEM((CHUNK,), jnp.int32),
            pltpu.VMEM((CHUNK,), jnp.int32),
            pltpu.VMEM((CHUNK, HID), jnp.float32),
            pltpu.VMEM((CHUNK, HID), jnp.float32),
            pltpu.SemaphoreType.DMA,
            pltpu.SemaphoreType.DMA,
            pltpu.VMEM_SHARED((NPAD, HID), jnp.float32),
        ],
    )(dstp, zf, ones_h)


def _sage1_body(p0, p1, c0, c1, x, wl, bl, wr, lnw, lnb, out):
    agg = (p0[...] + p1[...]) / jnp.maximum(c0[...] + c1[...], 1.0)
    y = (jnp.dot(agg, wl[...], preferred_element_type=jnp.float32)
         + bl[...]
         + jnp.dot(x[...], wr[...], preferred_element_type=jnp.float32))
    h1 = jnp.maximum(y, 0.0)
    mu = jnp.mean(h1, axis=1, keepdims=True)
    var = jnp.mean((h1 - mu) * (h1 - mu), axis=1, keepdims=True)
    x1 = (h1 - mu) * lax.rsqrt(var + 1e-5) * lnw[...] + lnb[...]
    out[...] = jnp.maximum(x1 + x[...], 0.0)


def _sage1_call(p0, p1, c0, c1, x, wlT, bl, wrT, lnw, lnb):
    spec_f = pl.BlockSpec((BLK, HID), lambda i: (i, 0))
    spec_w = pl.BlockSpec((HID, HID), lambda i: (0, 0))
    spec_b = pl.BlockSpec((1, HID), lambda i: (0, 0))
    return pl.pallas_call(
        _sage1_body,
        grid=(NPAD // BLK,),
        in_specs=[spec_f, spec_f, spec_f, spec_f, spec_f, spec_w, spec_b,
                  spec_w, spec_b, spec_b],
        out_specs=spec_f,
        out_shape=jax.ShapeDtypeStruct((NPAD, HID), jnp.float32),
    )(p0, p1, c0, c1, x, wlT, bl, wrT, lnw, lnb)


def _sage2_body(p0, p1, c0, c1, x1, res, wl, bl, wr, pw, pb, out):
    agg = (p0[...] + p1[...]) / jnp.maximum(c0[...] + c1[...], 1.0)
    y = (jnp.dot(agg, wl[...], preferred_element_type=jnp.float32)
         + bl[...]
         + jnp.dot(x1[...], wr[...], preferred_element_type=jnp.float32))
    xo = jnp.maximum(y + res[...], 0.0)
    o = jnp.dot(xo, pw[...], preferred_element_type=jnp.float32) + pb[...]
    rowid = (pl.program_id(0) * BLK
             + lax.broadcasted_iota(jnp.int32, (BLK, OUTP), 0))
    out[...] = jnp.where(rowid < N, o, 0.0)


def _sage2_call(p0, p1, c0, c1, x1, res, wl2T, bl2, wr2T, projWT, projb):
    spec_f = pl.BlockSpec((BLK, HID), lambda i: (i, 0))
    spec_w = pl.BlockSpec((HID, HID), lambda i: (0, 0))
    spec_b = pl.BlockSpec((1, HID), lambda i: (0, 0))
    return pl.pallas_call(
        _sage2_body,
        grid=(NPAD // BLK,),
        in_specs=[spec_f, spec_f, spec_f, spec_f, spec_f, spec_f, spec_w,
                  spec_b, spec_w,
                  pl.BlockSpec((HID, OUTP), lambda i: (0, 0)),
                  pl.BlockSpec((1, OUTP), lambda i: (0, 0))],
        out_specs=pl.BlockSpec((BLK, OUTP), lambda i: (i, 0)),
        out_shape=jax.ShapeDtypeStruct((NPAD, OUTP), jnp.float32),
    )(p0, p1, c0, c1, x1, res, wl2T, bl2, wr2T, projWT, projb)


KB_NCH = KB_PT // KB_CH


def _kb_sum(rows, out_v):
    for t in range(KB_CH):
        for half in range(2):
            acc = rows[t * BOM, pl.ds(half * 16, 16)]
            for j in range(1, BOM):
                acc = acc + rows[t * BOM + j, pl.ds(half * 16, 16)]
            out_v[t, pl.ds(half * 16, 16)] = acc


def _kb_body(proj, kbf, out, kidx, rows0, rows1, out_v, sem0, sem1):
    cid = lax.axis_index("c")
    sid = lax.axis_index("s")
    nch = jnp.where(cid == 0, KB0, KB1)
    tbase = pl.multiple_of(
        jnp.where(cid == 0, sid * KB0, 16 * KB0 + sid * KB1) * KB_CH, KB_CH)
    ibase = pl.multiple_of(tbase * BOM, KB_CH * BOM)
    pltpu.sync_copy(kbf.at[pl.ds(ibase, KBMX * KB_CH * BOM)], kidx)

    def emit(c, rows):
        _kb_sum(rows, out_v)
        toff = pl.multiple_of(tbase + c * KB_CH, KB_CH)
        pltpu.sync_copy(out_v, out.at[pl.ds(toff, KB_CH)])

    def gidx(c):
        return kidx.at[pl.ds(c * KB_CH * BOM, KB_CH * BOM)]

    pltpu.async_copy(proj.at[gidx(0)], rows0, sem0)

    def pair(i, carry):
        a = 2 * i
        pltpu.async_copy(proj.at[gidx(a + 1)], rows1, sem1)
        pltpu.make_async_copy(proj.at[gidx(a)], rows0, sem0).wait()
        emit(a, rows0)

        @pl.when(a + 2 < nch)
        def _():
            pltpu.async_copy(proj.at[gidx(a + 2)], rows0, sem0)

        pltpu.make_async_copy(proj.at[gidx(a + 1)], rows1, sem1).wait()
        emit(a + 1, rows1)
        return carry

    lax.fori_loop(0, nch // 2, pair, 0)


def _kb_call(proj, kbf):
    return pl.kernel(
        _kb_body,
        out_type=jax.ShapeDtypeStruct((NPAD, OUTP), jnp.float32),
        mesh=plsc.VectorSubcoreMesh(core_axis_name="c", subcore_axis_name="s",
                                    num_cores=2, num_subcores=16),
        scratch_types=[
            pltpu.VMEM((KBMX * KB_CH * BOM,), jnp.int32),
            pltpu.VMEM((KB_CH * BOM, OUTP), jnp.float32),
            pltpu.VMEM((KB_CH * BOM, OUTP), jnp.float32),
            pltpu.VMEM((KB_CH, OUTP), jnp.float32),
            pltpu.SemaphoreType.DMA,
            pltpu.SemaphoreType.DMA,
        ],
    )(proj, kbf)


def kernel(x_key, keybom, key_aggregation_status, edge_index,
           lstm_Wih, lstm_Whh, lstm_bih, lstm_bhh,
           sage1_Wl, sage1_bl, sage1_Wr, ln1_w, ln1_b,
           sage2_Wl, sage2_bl, sage2_Wr, proj_W, proj_b):
    f32 = jnp.float32
    x_t = jnp.zeros((TPAD, NPAD), f32).at[:T_IN, :N].set(x_key.T)
    wih_row = lstm_Wih.reshape(1, G4)
    whhT = lstm_Whh.T.astype(jnp.bfloat16)
    bias = (lstm_bih + lstm_bhh).reshape(1, G4)

    src = edge_index[0]
    dst = edge_index[1]
    srcp = jnp.full((EPAD,), N, jnp.int32).at[:E].set(src)
    dstp = jnp.full((EPAD,), N, jnp.int32).at[:E].set(dst)
    zf = jnp.zeros((NPAD, HID), f32)
    ones_h = jnp.ones((CHUNK, HID), f32)

    h = _lstm_call(x_t, wih_row, whhT, bias)
    res = h
    cnt = _cnt_call(dstp, zf, ones_h)

    part1 = _agg_call(h, srcp, dstp, zf)
    x1 = _sage1_call(part1[0], part1[1], cnt[0], cnt[1], h,
                     sage1_Wl.T, sage1_bl.reshape(1, HID), sage1_Wr.T,
                     ln1_w.reshape(1, HID), ln1_b.reshape(1, HID))

    part2 = _agg_call(x1, srcp, dstp, zf)
    projWT = jnp.zeros((HID, OUTP), f32).at[:, :OUT_CH].set(proj_W.T)
    projb = jnp.zeros((1, OUTP), f32).at[0, :OUT_CH].set(proj_b)
    po = _sage2_call(part2[0], part2[1], cnt[0], cnt[1], x1, res,
                     sage2_Wl.T, sage2_bl.reshape(1, HID), sage2_Wr.T,
                     projWT, projb)

    kb = jnp.where(keybom < 0, N, keybom)
    kbf = jnp.zeros((KBF_LEN,), jnp.int32).at[:N * BOM].set(kb.reshape(-1))
    ko = _kb_call(po, kbf)
    return ko[:N, :OUT_CH].reshape(N, TIME_STEPS, N_QUANTILES)


# speedup vs baseline: 1.1411x; 1.0240x over previous
import jax
import jax.numpy as jnp
from jax import lax
from jax.experimental import pallas as pl
from jax.experimental.pallas import tpu as pltpu
from jax.experimental.pallas import tpu_sc as plsc

N = 10000
NPAD = 10240
HID = 128
T_IN = 30
TPAD = 32
G4 = 4 * HID
OUT_CH = 21
OUTP = 128
TIME_STEPS = 7
N_QUANTILES = 3
E = 320000
CHUNK = 128
NCHUNK = 79
EPT = NCHUNK * CHUNK
EPAD = 32 * EPT
ROWS_PER_TILE = NPAD // 16
BOM = 10
KB_PT = NPAD // 32
KB_CH = 8
AK0, AK1 = 111, 47
KB0, KB1 = 72, 8
KBMX = max(KB0, KB1)
KBF_LEN = NPAD * BOM + KBMX * KB_CH * BOM
BLK = 1024


def _lstm_body(xt_ref, wih_ref, whh_ref, b_ref, out_ref):
    blk = out_ref.shape[0]

    def step(t, carry):
        h, c = carry
        xt = xt_ref[t, :][:, None]
        gates = (xt * wih_ref[...]
                 + jnp.dot(h.astype(jnp.bfloat16), whh_ref[...],
                           preferred_element_type=jnp.float32)
                 + b_ref[...])
        i = jax.nn.sigmoid(gates[:, 0 * HID:1 * HID])
        f = jax.nn.sigmoid(gates[:, 1 * HID:2 * HID])
        g = jnp.tanh(gates[:, 2 * HID:3 * HID])
        o = jax.nn.sigmoid(gates[:, 3 * HID:4 * HID])
        c = f * c + i * g
        h = o * jnp.tanh(c)
        return (h, c)

    h0 = jnp.zeros((blk, HID), jnp.float32)
    c0 = jnp.zeros((blk, HID), jnp.float32)
    h, _ = lax.fori_loop(0, T_IN, step, (h0, c0))
    out_ref[...] = h


def _lstm_call(x_t, wih_row, whhT, bias):
    return pl.pallas_call(
        _lstm_body,
        grid=(NPAD // BLK,),
        in_specs=[
            pl.BlockSpec((TPAD, BLK), lambda i: (0, i)),
            pl.BlockSpec((1, G4), lambda i: (0, 0)),
            pl.BlockSpec((HID, G4), lambda i: (0, 0)),
            pl.BlockSpec((1, G4), lambda i: (0, 0)),
        ],
        out_specs=pl.BlockSpec((BLK, HID), lambda i: (i, 0)),
        out_shape=jax.ShapeDtypeStruct((NPAD, HID), jnp.float32),
    )(x_t, wih_row, whhT, bias)


def _agg_body(feat, srcp, dstp, zf, part,
              sidx0, sidx1, didx0, didx1, rows0, rows1,
              semi0, semi1, semg0, semg1, acc_sh):
    cid = lax.axis_index("c")
    sid = lax.axis_index("s")
    wid = cid * 16 + sid
    r0 = pl.multiple_of(sid * ROWS_PER_TILE, ROWS_PER_TILE)
    for k in range(ROWS_PER_TILE // CHUNK):
        rk = pl.multiple_of(r0 + k * CHUNK, CHUNK)
        pltpu.sync_copy(zf.at[pl.ds(rk, CHUNK)], rows0)
        pltpu.sync_copy(rows0, acc_sh.at[pl.ds(rk, CHUNK)])
    plsc.subcore_barrier()

    nchunk = jnp.where(cid == 0, AK0, AK1)
    base = pl.multiple_of(
        jnp.where(cid == 0, sid * AK0, 16 * AK0 + sid * AK1) * CHUNK, CHUNK)

    def idx_start(c, sb, db, semi):
        off = pl.multiple_of(base + c * CHUNK, CHUNK)
        pltpu.async_copy(srcp.at[pl.ds(off, CHUNK)], sb, semi)
        pltpu.async_copy(dstp.at[pl.ds(off, CHUNK)], db, semi)

    def idx_wait(c, sb, db, semi):
        off = pl.multiple_of(base + c * CHUNK, CHUNK)
        pltpu.make_async_copy(srcp.at[pl.ds(off, CHUNK)], sb, semi).wait()
        pltpu.make_async_copy(dstp.at[pl.ds(off, CHUNK)], db, semi).wait()

    idx_start(0, sidx0, didx0, semi0)
    idx_start(1, sidx1, didx1, semi1)
    idx_wait(0, sidx0, didx0, semi0)
    pltpu.async_copy(feat.at[sidx0], rows0, semg0)
    idx_wait(1, sidx1, didx1, semi1)

    def pair(i, carry):
        a = 2 * i
        pltpu.async_copy(feat.at[sidx1], rows1, semg1)
        pltpu.make_async_copy(feat.at[sidx0], rows0, semg0).wait()
        pltpu.sync_copy(rows0, acc_sh.at[didx0], add=True)
        idx_start(a + 2, sidx0, didx0, semi0)
        pltpu.make_async_copy(feat.at[sidx1], rows1, semg1).wait()
        pltpu.sync_copy(rows1, acc_sh.at[didx1], add=True)

        @pl.when(a + 3 < nchunk)
        def _():
            idx_start(a + 3, sidx1, didx1, semi1)

        idx_wait(a + 2, sidx0, didx0, semi0)

        @pl.when(a + 3 < nchunk)
        def _():
            idx_wait(a + 3, sidx1, didx1, semi1)

        pltpu.async_copy(feat.at[sidx0], rows0, semg0)
        return carry

    lax.fori_loop(0, (nchunk - 1) // 2, pair, 0)
    pltpu.make_async_copy(feat.at[sidx0], rows0, semg0).wait()
    pltpu.sync_copy(rows0, acc_sh.at[didx0], add=True)
    plsc.subcore_barrier()
    for k in range(ROWS_PER_TILE // CHUNK):
        rk = pl.multiple_of(r0 + k * CHUNK, CHUNK)
        pltpu.sync_copy(acc_sh.at[pl.ds(rk, CHUNK)], rows0)
        pltpu.sync_copy(rows0, part.at[cid, pl.ds(rk, CHUNK)])


def _agg_call(feat, srcp, dstp, zf):
    return pl.kernel(
        _agg_body,
        out_type=jax.ShapeDtypeStruct((2, NPAD, HID), jnp.float32),
        mesh=plsc.VectorSubcoreMesh(core_axis_name="c", subcore_axis_name="s",
                                    num_cores=2, num_subcores=16),
        scratch_types=[
            pltpu.VMEM((CHUNK,), jnp.int32),
            pltpu.VMEM((CHUNK,), jnp.int32),
            pltpu.VMEM((CHUNK,), jnp.int32),
            pltpu.VMEM((CHUNK,), jnp.int32),
            pltpu.VMEM((CHUNK, HID), jnp.float32),
            pltpu.VMEM((CHUNK, HID), jnp.float32),
            pltpu.SemaphoreType.DMA,
            pltpu.SemaphoreType.DMA,
            pltpu.SemaphoreType.DMA,
            pltpu.SemaphoreType.DMA,
            pltpu.VMEM_SHARED((NPAD, HID), jnp.float32),
        ],
    )(feat, srcp, dstp, zf)


def _cnt_body(dstp, zf, ones_h, cntp,
              didx0, didx1, ones_v, rows_v, semi0, semi1, acc_sh):
    cid = lax.axis_index("c")
    sid = lax.axis_index("s")
    wid = cid * 16 + sid
    r0 = pl.multiple_of(sid * ROWS_PER_TILE, ROWS_PER_TILE)
    for k in range(ROWS_PER_TILE // CHUNK):
        rk = pl.multiple_of(r0 + k * CHUNK, CHUNK)
        pltpu.sync_copy(zf.at[pl.ds(rk, CHUNK)], rows_v)
        pltpu.sync_copy(rows_v, acc_sh.at[pl.ds(rk, CHUNK)])
    pltpu.sync_copy(ones_h, ones_v)
    base = wid * EPT

    def idx_start(c, db, semi):
        off = pl.multiple_of(base + c * CHUNK, CHUNK)
        pltpu.async_copy(dstp.at[pl.ds(off, CHUNK)], db, semi)

    def idx_wait(c, db, semi):
        off = pl.multiple_of(base + c * CHUNK, CHUNK)
        pltpu.make_async_copy(dstp.at[pl.ds(off, CHUNK)], db, semi).wait()

    idx_start(0, didx0, semi0)
    idx_start(1, didx1, semi1)
    plsc.subcore_barrier()

    def pair(i, carry):
        a = 2 * i
        idx_wait(a, didx0, semi0)
        pltpu.sync_copy(ones_v, acc_sh.at[didx0], add=True)
        idx_start(a + 2, didx0, semi0)
        idx_wait(a + 1, didx1, semi1)
        pltpu.sync_copy(ones_v, acc_sh.at[didx1], add=True)

        @pl.when(a + 3 < NCHUNK)
        def _():
            idx_start(a + 3, didx1, semi1)

        return carry

    lax.fori_loop(0, (NCHUNK - 1) // 2, pair, 0)
    idx_wait(NCHUNK - 1, didx0, semi0)
    pltpu.sync_copy(ones_v, acc_sh.at[didx0], add=True)
    plsc.subcore_barrier()
    for k in range(ROWS_PER_TILE // CHUNK):
        rk = pl.multiple_of(r0 + k * CHUNK, CHUNK)
        pltpu.sync_copy(acc_sh.at[pl.ds(rk, CHUNK)], rows_v)
        pltpu.sync_copy(rows_v, cntp.at[cid, pl.ds(rk, CHUNK)])


def _cnt_call(dstp, zf, ones_h):
    return pl.kernel(
        _cnt_body,
        out_type=jax.ShapeDtypeStruct((2, NPAD, HID), jnp.float32),
        mesh=plsc.VectorSubcoreMesh(core_axis_name="c", subcore_axis_name="s",
                                    num_cores=2, num_subcores=16),
        scratch_types=[
            pltpu.VMEM((CHUNK,), jnp.int32),
            pltpu.VMEM((CHUNK,), jnp.int32),
            pltpu.VMEM((CHUNK, HID), jnp.float32),
            pltpu.VMEM((CHUNK, HID), jnp.float32),
            pltpu.SemaphoreType.DMA,
            pltpu.SemaphoreType.DMA,
            pltpu.VMEM_SHARED((NPAD, HID), jnp.float32),
        ],
    )(dstp, zf, ones_h)


def _sage1_body(p0, p1, c0, c1, x, wl, bl, wr, lnw, lnb, out):
    agg = (p0[...] + p1[...]) / jnp.maximum(c0[...] + c1[...], 1.0)
    y = (jnp.dot(agg, wl[...], preferred_element_type=jnp.float32)
         + bl[...]
         + jnp.dot(x[...], wr[...], preferred_element_type=jnp.float32))
    h1 = jnp.maximum(y, 0.0)
    mu = jnp.mean(h1, axis=1, keepdims=True)
    var = jnp.mean((h1 - mu) * (h1 - mu), axis=1, keepdims=True)
    x1 = (h1 - mu) * lax.rsqrt(var + 1e-5) * lnw[...] + lnb[...]
    out[...] = jnp.maximum(x1 + x[...], 0.0)


def _sage1_call(p0, p1, c0, c1, x, wlT, bl, wrT, lnw, lnb):
    spec_f = pl.BlockSpec((BLK, HID), lambda i: (i, 0))
    spec_w = pl.BlockSpec((HID, HID), lambda i: (0, 0))
    spec_b = pl.BlockSpec((1, HID), lambda i: (0, 0))
    return pl.pallas_call(
        _sage1_body,
        grid=(NPAD // BLK,),
        in_specs=[spec_f, spec_f, spec_f, spec_f, spec_f, spec_w, spec_b,
                  spec_w, spec_b, spec_b],
        out_specs=spec_f,
        out_shape=jax.ShapeDtypeStruct((NPAD, HID), jnp.float32),
    )(p0, p1, c0, c1, x, wlT, bl, wrT, lnw, lnb)


def _sage2_body(p0, p1, c0, c1, x1, res, wl, bl, wr, pw, pb, out):
    agg = (p0[...] + p1[...]) / jnp.maximum(c0[...] + c1[...], 1.0)
    y = (jnp.dot(agg, wl[...], preferred_element_type=jnp.float32)
         + bl[...]
         + jnp.dot(x1[...], wr[...], preferred_element_type=jnp.float32))
    xo = jnp.maximum(y + res[...], 0.0)
    o = jnp.dot(xo, pw[...], preferred_element_type=jnp.float32) + pb[...]
    rowid = (pl.program_id(0) * BLK
             + lax.broadcasted_iota(jnp.int32, (BLK, OUTP), 0))
    out[...] = jnp.where(rowid < N, o, 0.0)


def _sage2_call(p0, p1, c0, c1, x1, res, wl2T, bl2, wr2T, projWT, projb):
    spec_f = pl.BlockSpec((BLK, HID), lambda i: (i, 0))
    spec_w = pl.BlockSpec((HID, HID), lambda i: (0, 0))
    spec_b = pl.BlockSpec((1, HID), lambda i: (0, 0))
    return pl.pallas_call(
        _sage2_body,
        grid=(NPAD // BLK,),
        in_specs=[spec_f, spec_f, spec_f, spec_f, spec_f, spec_f, spec_w,
                  spec_b, spec_w,
                  pl.BlockSpec((HID, OUTP), lambda i: (0, 0)),
                  pl.BlockSpec((1, OUTP), lambda i: (0, 0))],
        out_specs=pl.BlockSpec((BLK, OUTP), lambda i: (i, 0)),
        out_shape=jax.ShapeDtypeStruct((NPAD, OUTP), jnp.float32),
    )(p0, p1, c0, c1, x1, res, wl2T, bl2, wr2T, projWT, projb)


KB_NCH = KB_PT // KB_CH


def _kb_sum(rows, out_v):
    for t in range(KB_CH):
        for half in range(2):
            acc = rows[t * BOM, pl.ds(half * 16, 16)]
            for j in range(1, BOM):
                acc = acc + rows[t * BOM + j, pl.ds(half * 16, 16)]
            out_v[t, pl.ds(half * 16, 16)] = acc


def _kb_body(proj, kbf, out, kidx, rows0, rows1, out_v, sem0, sem1):
    cid = lax.axis_index("c")
    sid = lax.axis_index("s")
    nch = jnp.where(cid == 0, KB0, KB1)
    tbase = pl.multiple_of(
        jnp.where(cid == 0, sid * KB0, 16 * KB0 + sid * KB1) * KB_CH, KB_CH)
    ibase = pl.multiple_of(tbase * BOM, KB_CH * BOM)
    pltpu.sync_copy(kbf.at[pl.ds(ibase, KBMX * KB_CH * BOM)], kidx)

    def emit(c, rows):
        _kb_sum(rows, out_v)
        toff = pl.multiple_of(tbase + c * KB_CH, KB_CH)
        pltpu.sync_copy(out_v, out.at[pl.ds(toff, KB_CH)])

    def gidx(c):
        return kidx.at[pl.ds(c * KB_CH * BOM, KB_CH * BOM)]

    pltpu.async_copy(proj.at[gidx(0)], rows0, sem0)

    def pair(i, carry):
        a = 2 * i
        pltpu.async_copy(proj.at[gidx(a + 1)], rows1, sem1)
        pltpu.make_async_copy(proj.at[gidx(a)], rows0, sem0).wait()
        emit(a, rows0)

        @pl.when(a + 2 < nch)
        def _():
            pltpu.async_copy(proj.at[gidx(a + 2)], rows0, sem0)

        pltpu.make_async_copy(proj.at[gidx(a + 1)], rows1, sem1).wait()
        emit(a + 1, rows1)
        return carry

    lax.fori_loop(0, nch // 2, pair, 0)


def _kb_call(proj, kbf):
    return pl.kernel(
        _kb_body,
        out_type=jax.ShapeDtypeStruct((NPAD, OUTP), jnp.float32),
        mesh=plsc.VectorSubcoreMesh(core_axis_name="c", subcore_axis_name="s",
                                    num_cores=2, num_subcores=16),
        scratch_types=[
            pltpu.VMEM((KBMX * KB_CH * BOM,), jnp.int32),
            pltpu.VMEM((KB_CH * BOM, OUTP), jnp.float32),
            pltpu.VMEM((KB_CH * BOM, OUTP), jnp.float32),
            pltpu.VMEM((KB_CH, OUTP), jnp.float32),
            pltpu.SemaphoreType.DMA,
            pltpu.SemaphoreType.DMA,
        ],
    )(proj, kbf)


def kernel(x_key, keybom, key_aggregation_status, edge_index,
           lstm_Wih, lstm_Whh, lstm_bih, lstm_bhh,
           sage1_Wl, sage1_bl, sage1_Wr, ln1_w, ln1_b,
           sage2_Wl, sage2_bl, sage2_Wr, proj_W, proj_b):
    f32 = jnp.float32
    x_t = jnp.zeros((TPAD, NPAD), f32).at[:T_IN, :N].set(x_key.T)
    wih_row = lstm_Wih.reshape(1, G4)
    whhT = lstm_Whh.T.astype(jnp.bfloat16)
    bias = (lstm_bih + lstm_bhh).reshape(1, G4)

    src = edge_index[0]
    dst = edge_index[1]
    srcp = jnp.full((EPAD,), N, jnp.int32).at[:E].set(src)
    dstp = jnp.full((EPAD,), N, jnp.int32).at[:E].set(dst)
    zf = jnp.zeros((NPAD, HID), f32)
    ones_h = jnp.ones((CHUNK, HID), f32)

    h = _lstm_call(x_t, wih_row, whhT, bias)
    res = h
    cnt = _cnt_call(dstp, zf, ones_h)

    part1 = _agg_call(h, srcp, dstp, zf)
    x1 = _sage1_call(part1[0], part1[1], cnt[0], cnt[1], h,
                     sage1_Wl.T, sage1_bl.reshape(1, HID), sage1_Wr.T,
                     ln1_w.reshape(1, HID), ln1_b.reshape(1, HID))

    part2 = _agg_call(x1, srcp, dstp, zf)
    projWT = jnp.zeros((HID, OUTP), f32).at[:, :OUT_CH].set(proj_W.T)
    projb = jnp.zeros((1, OUTP), f32).at[0, :OUT_CH].set(proj_b)
    po = _sage2_call(part2[0], part2[1], cnt[0], cnt[1], x1, res,
                     sage2_Wl.T, sage2_bl.reshape(1, HID), sage2_Wr.T,
                     projWT, projb)

    kb = jnp.where(keybom < 0, N, keybom)
    kbf = jnp.zeros((KBF_LEN,), jnp.int32).at[:N * BOM].set(kb.reshape(-1))
    ko = _kb_call(po, kbf)
    return ko[:N, :OUT_CH].reshape(N, TIME_STEPS, N_QUANTILES)


# speedup vs baseline: 1.1658x; 1.0217x over previous
import jax
import jax.numpy as jnp
from jax import lax
from jax.experimental import pallas as pl
from jax.experimental.pallas import tpu as pltpu
from jax.experimental.pallas import tpu_sc as plsc

N = 10000
NPAD = 10240
HID = 128
T_IN = 30
TPAD = 32
G4 = 4 * HID
OUT_CH = 21
OUTP = 128
TIME_STEPS = 7
N_QUANTILES = 3
E = 320000
CHUNK = 128
NCHUNK = 79
EPT = NCHUNK * CHUNK
EPAD = 32 * EPT
ROWS_PER_TILE = NPAD // 16
BOM = 10
KB_PT = NPAD // 32
KB_CH = 8
AK0, AK1 = 111, 47
KB0, KB1 = 72, 8
KBMX = max(KB0, KB1)
KBF_LEN = NPAD * BOM + KBMX * KB_CH * BOM
BLK = 2048


def _lstm_body(xt_ref, wih_ref, whh_ref, b_ref, out_ref):
    blk = out_ref.shape[0]

    def step(t, carry):
        h, c = carry
        xt = xt_ref[t, :][:, None]
        gates = (xt * wih_ref[...]
                 + jnp.dot(h.astype(jnp.bfloat16), whh_ref[...],
                           preferred_element_type=jnp.float32)
                 + b_ref[...])
        i = jax.nn.sigmoid(gates[:, 0 * HID:1 * HID])
        f = jax.nn.sigmoid(gates[:, 1 * HID:2 * HID])
        g = jnp.tanh(gates[:, 2 * HID:3 * HID])
        o = jax.nn.sigmoid(gates[:, 3 * HID:4 * HID])
        c = f * c + i * g
        h = o * jnp.tanh(c)
        return (h, c)

    h0 = jnp.zeros((blk, HID), jnp.float32)
    c0 = jnp.zeros((blk, HID), jnp.float32)
    h, _ = lax.fori_loop(0, T_IN, step, (h0, c0))
    out_ref[...] = h


def _lstm_call(x_t, wih_row, whhT, bias):
    return pl.pallas_call(
        _lstm_body,
        grid=(NPAD // BLK,),
        in_specs=[
            pl.BlockSpec((TPAD, BLK), lambda i: (0, i)),
            pl.BlockSpec((1, G4), lambda i: (0, 0)),
            pl.BlockSpec((HID, G4), lambda i: (0, 0)),
            pl.BlockSpec((1, G4), lambda i: (0, 0)),
        ],
        out_specs=pl.BlockSpec((BLK, HID), lambda i: (i, 0)),
        out_shape=jax.ShapeDtypeStruct((NPAD, HID), jnp.float32),
    )(x_t, wih_row, whhT, bias)


def _agg_body(feat, srcp, dstp, zf, part,
              sidx0, sidx1, didx0, didx1, rows0, rows1,
              semi0, semi1, semg0, semg1, acc_sh):
    cid = lax.axis_index("c")
    sid = lax.axis_index("s")
    wid = cid * 16 + sid
    r0 = pl.multiple_of(sid * ROWS_PER_TILE, ROWS_PER_TILE)
    for k in range(ROWS_PER_TILE // CHUNK):
        rk = pl.multiple_of(r0 + k * CHUNK, CHUNK)
        pltpu.sync_copy(zf.at[pl.ds(rk, CHUNK)], rows0)
        pltpu.sync_copy(rows0, acc_sh.at[pl.ds(rk, CHUNK)])
    plsc.subcore_barrier()

    nchunk = jnp.where(cid == 0, AK0, AK1)
    base = pl.multiple_of(
        jnp.where(cid == 0, sid * AK0, 16 * AK0 + sid * AK1) * CHUNK, CHUNK)

    def idx_start(c, sb, db, semi):
        off = pl.multiple_of(base + c * CHUNK, CHUNK)
        pltpu.async_copy(srcp.at[pl.ds(off, CHUNK)], sb, semi)
        pltpu.async_copy(dstp.at[pl.ds(off, CHUNK)], db, semi)

    def idx_wait(c, sb, db, semi):
        off = pl.multiple_of(base + c * CHUNK, CHUNK)
        pltpu.make_async_copy(srcp.at[pl.ds(off, CHUNK)], sb, semi).wait()
        pltpu.make_async_copy(dstp.at[pl.ds(off, CHUNK)], db, semi).wait()

    idx_start(0, sidx0, didx0, semi0)
    idx_start(1, sidx1, didx1, semi1)
    idx_wait(0, sidx0, didx0, semi0)
    pltpu.async_copy(feat.at[sidx0], rows0, semg0)
    idx_wait(1, sidx1, didx1, semi1)

    def pair(i, carry):
        a = 2 * i
        pltpu.async_copy(feat.at[sidx1], rows1, semg1)
        pltpu.make_async_copy(feat.at[sidx0], rows0, semg0).wait()
        pltpu.sync_copy(rows0, acc_sh.at[didx0], add=True)
        idx_start(a + 2, sidx0, didx0, semi0)
        pltpu.make_async_copy(feat.at[sidx1], rows1, semg1).wait()
        pltpu.sync_copy(rows1, acc_sh.at[didx1], add=True)

        @pl.when(a + 3 < nchunk)
        def _():
            idx_start(a + 3, sidx1, didx1, semi1)

        idx_wait(a + 2, sidx0, didx0, semi0)

        @pl.when(a + 3 < nchunk)
        def _():
            idx_wait(a + 3, sidx1, didx1, semi1)

        pltpu.async_copy(feat.at[sidx0], rows0, semg0)
        return carry

    lax.fori_loop(0, (nchunk - 1) // 2, pair, 0)
    pltpu.make_async_copy(feat.at[sidx0], rows0, semg0).wait()
    pltpu.sync_copy(rows0, acc_sh.at[didx0], add=True)
    plsc.subcore_barrier()
    for k in range(ROWS_PER_TILE // CHUNK):
        rk = pl.multiple_of(r0 + k * CHUNK, CHUNK)
        pltpu.sync_copy(acc_sh.at[pl.ds(rk, CHUNK)], rows0)
        pltpu.sync_copy(rows0, part.at[cid, pl.ds(rk, CHUNK)])


def _agg_call(feat, srcp, dstp, zf):
    return pl.kernel(
        _agg_body,
        out_type=jax.ShapeDtypeStruct((2, NPAD, HID), jnp.float32),
        mesh=plsc.VectorSubcoreMesh(core_axis_name="c", subcore_axis_name="s",
                                    num_cores=2, num_subcores=16),
        scratch_types=[
            pltpu.VMEM((CHUNK,), jnp.int32),
            pltpu.VMEM((CHUNK,), jnp.int32),
            pltpu.VMEM((CHUNK,), jnp.int32),
            pltpu.VMEM((CHUNK,), jnp.int32),
            pltpu.VMEM((CHUNK, HID), jnp.float32),
            pltpu.VMEM((CHUNK, HID), jnp.float32),
            pltpu.SemaphoreType.DMA,
            pltpu.SemaphoreType.DMA,
            pltpu.SemaphoreType.DMA,
            pltpu.SemaphoreType.DMA,
            pltpu.VMEM_SHARED((NPAD, HID), jnp.float32),
        ],
    )(feat, srcp, dstp, zf)


def _cnt_body(dstp, zf, ones_h, cntp,
              didx0, didx1, ones_v, rows_v, semi0, semi1, acc_sh):
    cid = lax.axis_index("c")
    sid = lax.axis_index("s")
    wid = cid * 16 + sid
    r0 = pl.multiple_of(sid * ROWS_PER_TILE, ROWS_PER_TILE)
    for k in range(ROWS_PER_TILE // CHUNK):
        rk = pl.multiple_of(r0 + k * CHUNK, CHUNK)
        pltpu.sync_copy(zf.at[pl.ds(rk, CHUNK)], rows_v)
        pltpu.sync_copy(rows_v, acc_sh.at[pl.ds(rk, CHUNK)])
    pltpu.sync_copy(ones_h, ones_v)
    base = wid * EPT

    def idx_start(c, db, semi):
        off = pl.multiple_of(base + c * CHUNK, CHUNK)
        pltpu.async_copy(dstp.at[pl.ds(off, CHUNK)], db, semi)

    def idx_wait(c, db, semi):
        off = pl.multiple_of(base + c * CHUNK, CHUNK)
        pltpu.make_async_copy(dstp.at[pl.ds(off, CHUNK)], db, semi).wait()

    idx_start(0, didx0, semi0)
    idx_start(1, didx1, semi1)
    plsc.subcore_barrier()

    def pair(i, carry):
        a = 2 * i
        idx_wait(a, didx0, semi0)
        pltpu.sync_copy(ones_v, acc_sh.at[didx0], add=True)
        idx_start(a + 2, didx0, semi0)
        idx_wait(a + 1, didx1, semi1)
        pltpu.sync_copy(ones_v, acc_sh.at[didx1], add=True)

        @pl.when(a + 3 < NCHUNK)
        def _():
            idx_start(a + 3, didx1, semi1)

        return carry

    lax.fori_loop(0, (NCHUNK - 1) // 2, pair, 0)
    idx_wait(NCHUNK - 1, didx0, semi0)
    pltpu.sync_copy(ones_v, acc_sh.at[didx0], add=True)
    plsc.subcore_barrier()
    for k in range(ROWS_PER_TILE // CHUNK):
        rk = pl.multiple_of(r0 + k * CHUNK, CHUNK)
        pltpu.sync_copy(acc_sh.at[pl.ds(rk, CHUNK)], rows_v)
        pltpu.sync_copy(rows_v, cntp.at[cid, pl.ds(rk, CHUNK)])


def _cnt_call(dstp, zf, ones_h):
    return pl.kernel(
        _cnt_body,
        out_type=jax.ShapeDtypeStruct((2, NPAD, HID), jnp.float32),
        mesh=plsc.VectorSubcoreMesh(core_axis_name="c", subcore_axis_name="s",
                                    num_cores=2, num_subcores=16),
        scratch_types=[
            pltpu.VMEM((CHUNK,), jnp.int32),
            pltpu.VMEM((CHUNK,), jnp.int32),
            pltpu.VMEM((CHUNK, HID), jnp.float32),
            pltpu.VMEM((CHUNK, HID), jnp.float32),
            pltpu.SemaphoreType.DMA,
            pltpu.SemaphoreType.DMA,
            pltpu.VMEM_SHARED((NPAD, HID), jnp.float32),
        ],
    )(dstp, zf, ones_h)


def _sage1_body(p0, p1, c0, c1, x, wl, bl, wr, lnw, lnb, out):
    agg = (p0[...] + p1[...]) / jnp.maximum(c0[...] + c1[...], 1.0)
    y = (jnp.dot(agg, wl[...], preferred_element_type=jnp.float32)
         + bl[...]
         + jnp.dot(x[...], wr[...], preferred_element_type=jnp.float32))
    h1 = jnp.maximum(y, 0.0)
    mu = jnp.mean(h1, axis=1, keepdims=True)
    var = jnp.mean((h1 - mu) * (h1 - mu), axis=1, keepdims=True)
    x1 = (h1 - mu) * lax.rsqrt(var + 1e-5) * lnw[...] + lnb[...]
    out[...] = jnp.maximum(x1 + x[...], 0.0)


def _sage1_call(p0, p1, c0, c1, x, wlT, bl, wrT, lnw, lnb):
    spec_f = pl.BlockSpec((BLK, HID), lambda i: (i, 0))
    spec_w = pl.BlockSpec((HID, HID), lambda i: (0, 0))
    spec_b = pl.BlockSpec((1, HID), lambda i: (0, 0))
    return pl.pallas_call(
        _sage1_body,
        grid=(NPAD // BLK,),
        in_specs=[spec_f, spec_f, spec_f, spec_f, spec_f, spec_w, spec_b,
                  spec_w, spec_b, spec_b],
        out_specs=spec_f,
        out_shape=jax.ShapeDtypeStruct((NPAD, HID), jnp.float32),
    )(p0, p1, c0, c1, x, wlT, bl, wrT, lnw, lnb)


def _sage2_body(p0, p1, c0, c1, x1, res, wl, bl, wr, pw, pb, out):
    agg = (p0[...] + p1[...]) / jnp.maximum(c0[...] + c1[...], 1.0)
    y = (jnp.dot(agg, wl[...], preferred_element_type=jnp.float32)
         + bl[...]
         + jnp.dot(x1[...], wr[...], preferred_element_type=jnp.float32))
    xo = jnp.maximum(y + res[...], 0.0)
    o = jnp.dot(xo, pw[...], preferred_element_type=jnp.float32) + pb[...]
    rowid = (pl.program_id(0) * BLK
             + lax.broadcasted_iota(jnp.int32, (BLK, OUTP), 0))
    out[...] = jnp.where(rowid < N, o, 0.0)


def _sage2_call(p0, p1, c0, c1, x1, res, wl2T, bl2, wr2T, projWT, projb):
    spec_f = pl.BlockSpec((BLK, HID), lambda i: (i, 0))
    spec_w = pl.BlockSpec((HID, HID), lambda i: (0, 0))
    spec_b = pl.BlockSpec((1, HID), lambda i: (0, 0))
    return pl.pallas_call(
        _sage2_body,
        grid=(NPAD // BLK,),
        in_specs=[spec_f, spec_f, spec_f, spec_f, spec_f, spec_f, spec_w,
                  spec_b, spec_w,
                  pl.BlockSpec((HID, OUTP), lambda i: (0, 0)),
                  pl.BlockSpec((1, OUTP), lambda i: (0, 0))],
        out_specs=pl.BlockSpec((BLK, OUTP), lambda i: (i, 0)),
        out_shape=jax.ShapeDtypeStruct((NPAD, OUTP), jnp.float32),
    )(p0, p1, c0, c1, x1, res, wl2T, bl2, wr2T, projWT, projb)


KB_NCH = KB_PT // KB_CH


def _kb_sum(rows, out_v):
    for t in range(KB_CH):
        for half in range(2):
            acc = rows[t * BOM, pl.ds(half * 16, 16)]
            for j in range(1, BOM):
                acc = acc + rows[t * BOM + j, pl.ds(half * 16, 16)]
            out_v[t, pl.ds(half * 16, 16)] = acc


def _kb_body(proj, kbf, out, kidx, rows0, rows1, out_v, sem0, sem1):
    cid = lax.axis_index("c")
    sid = lax.axis_index("s")
    nch = jnp.where(cid == 0, KB0, KB1)
    tbase = pl.multiple_of(
        jnp.where(cid == 0, sid * KB0, 16 * KB0 + sid * KB1) * KB_CH, KB_CH)
    ibase = pl.multiple_of(tbase * BOM, KB_CH * BOM)
    pltpu.sync_copy(kbf.at[pl.ds(ibase, KBMX * KB_CH * BOM)], kidx)

    def emit(c, rows):
        _kb_sum(rows, out_v)
        toff = pl.multiple_of(tbase + c * KB_CH, KB_CH)
        pltpu.sync_copy(out_v, out.at[pl.ds(toff, KB_CH)])

    def gidx(c):
        return kidx.at[pl.ds(c * KB_CH * BOM, KB_CH * BOM)]

    pltpu.async_copy(proj.at[gidx(0)], rows0, sem0)

    def pair(i, carry):
        a = 2 * i
        pltpu.async_copy(proj.at[gidx(a + 1)], rows1, sem1)
        pltpu.make_async_copy(proj.at[gidx(a)], rows0, sem0).wait()
        emit(a, rows0)

        @pl.when(a + 2 < nch)
        def _():
            pltpu.async_copy(proj.at[gidx(a + 2)], rows0, sem0)

        pltpu.make_async_copy(proj.at[gidx(a + 1)], rows1, sem1).wait()
        emit(a + 1, rows1)
        return carry

    lax.fori_loop(0, nch // 2, pair, 0)


def _kb_call(proj, kbf):
    return pl.kernel(
        _kb_body,
        out_type=jax.ShapeDtypeStruct((NPAD, OUTP), jnp.float32),
        mesh=plsc.VectorSubcoreMesh(core_axis_name="c", subcore_axis_name="s",
                                    num_cores=2, num_subcores=16),
        scratch_types=[
            pltpu.VMEM((KBMX * KB_CH * BOM,), jnp.int32),
            pltpu.VMEM((KB_CH * BOM, OUTP), jnp.float32),
            pltpu.VMEM((KB_CH * BOM, OUTP), jnp.float32),
            pltpu.VMEM((KB_CH, OUTP), jnp.float32),
            pltpu.SemaphoreType.DMA,
            pltpu.SemaphoreType.DMA,
        ],
    )(proj, kbf)


def kernel(x_key, keybom, key_aggregation_status, edge_index,
           lstm_Wih, lstm_Whh, lstm_bih, lstm_bhh,
           sage1_Wl, sage1_bl, sage1_Wr, ln1_w, ln1_b,
           sage2_Wl, sage2_bl, sage2_Wr, proj_W, proj_b):
    f32 = jnp.float32
    x_t = jnp.zeros((TPAD, NPAD), f32).at[:T_IN, :N].set(x_key.T)
    wih_row = lstm_Wih.reshape(1, G4)
    whhT = lstm_Whh.T.astype(jnp.bfloat16)
    bias = (lstm_bih + lstm_bhh).reshape(1, G4)

    src = edge_index[0]
    dst = edge_index[1]
    srcp = jnp.full((EPAD,), N, jnp.int32).at[:E].set(src)
    dstp = jnp.full((EPAD,), N, jnp.int32).at[:E].set(dst)
    zf = jnp.zeros((NPAD, HID), f32)
    ones_h = jnp.ones((CHUNK, HID), f32)

    h = _lstm_call(x_t, wih_row, whhT, bias)
    res = h
    cnt = _cnt_call(dstp, zf, ones_h)

    part1 = _agg_call(h, srcp, dstp, zf)
    x1 = _sage1_call(part1[0], part1[1], cnt[0], cnt[1], h,
                     sage1_Wl.T, sage1_bl.reshape(1, HID), sage1_Wr.T,
                     ln1_w.reshape(1, HID), ln1_b.reshape(1, HID))

    part2 = _agg_call(x1, srcp, dstp, zf)
    projWT = jnp.zeros((HID, OUTP), f32).at[:, :OUT_CH].set(proj_W.T)
    projb = jnp.zeros((1, OUTP), f32).at[0, :OUT_CH].set(proj_b)
    po = _sage2_call(part2[0], part2[1], cnt[0], cnt[1], x1, res,
                     sage2_Wl.T, sage2_bl.reshape(1, HID), sage2_Wr.T,
                     projWT, projb)

    kb = jnp.where(keybom < 0, N, keybom)
    kbf = jnp.zeros((KBF_LEN,), jnp.int32).at[:N * BOM].set(kb.reshape(-1))
    ko = _kb_call(po, kbf)
    return ko[:N, :OUT_CH].reshape(N, TIME_STEPS, N_QUANTILES)


# speedup vs baseline: 1.1690x; 1.0028x over previous
import jax
import jax.numpy as jnp
from jax import lax
from jax.experimental import pallas as pl
from jax.experimental.pallas import tpu as pltpu
from jax.experimental.pallas import tpu_sc as plsc

N = 10000
NPAD = 10240
HID = 128
T_IN = 30
TPAD = 32
G4 = 4 * HID
OUT_CH = 21
OUTP = 128
TIME_STEPS = 7
N_QUANTILES = 3
E = 320000
CHUNK = 128
NCHUNK = 79
EPT = NCHUNK * CHUNK
EPAD = 32 * EPT
ROWS_PER_TILE = NPAD // 16
BOM = 10
KB_PT = NPAD // 32
KB_CH = 8
AK0, AK1 = 111, 47
KB0, KB1 = 72, 8
KBMX = max(KB0, KB1)
KBF_LEN = NPAD * BOM + KBMX * KB_CH * BOM
BLK = 2560


def _lstm_body(xt_ref, wih_ref, whh_ref, b_ref, out_ref):
    blk = out_ref.shape[0]

    def step(t, carry):
        h, c = carry
        xt = xt_ref[t, :][:, None]
        gates = (xt * wih_ref[...]
                 + jnp.dot(h.astype(jnp.bfloat16), whh_ref[...],
                           preferred_element_type=jnp.float32)
                 + b_ref[...])
        i = jax.nn.sigmoid(gates[:, 0 * HID:1 * HID])
        f = jax.nn.sigmoid(gates[:, 1 * HID:2 * HID])
        g = jnp.tanh(gates[:, 2 * HID:3 * HID])
        o = jax.nn.sigmoid(gates[:, 3 * HID:4 * HID])
        c = f * c + i * g
        h = o * jnp.tanh(c)
        return (h, c)

    h0 = jnp.zeros((blk, HID), jnp.float32)
    c0 = jnp.zeros((blk, HID), jnp.float32)
    h, _ = lax.fori_loop(0, T_IN, step, (h0, c0))
    out_ref[...] = h


def _lstm_call(x_t, wih_row, whhT, bias):
    return pl.pallas_call(
        _lstm_body,
        grid=(NPAD // BLK,),
        in_specs=[
            pl.BlockSpec((TPAD, BLK), lambda i: (0, i)),
            pl.BlockSpec((1, G4), lambda i: (0, 0)),
            pl.BlockSpec((HID, G4), lambda i: (0, 0)),
            pl.BlockSpec((1, G4), lambda i: (0, 0)),
        ],
        out_specs=pl.BlockSpec((BLK, HID), lambda i: (i, 0)),
        out_shape=jax.ShapeDtypeStruct((NPAD, HID), jnp.float32),
    )(x_t, wih_row, whhT, bias)


def _agg_body(feat, srcp, dstp, zf, part,
              sidx0, sidx1, didx0, didx1, rows0, rows1,
              semi0, semi1, semg0, semg1, acc_sh):
    cid = lax.axis_index("c")
    sid = lax.axis_index("s")
    wid = cid * 16 + sid
    r0 = pl.multiple_of(sid * ROWS_PER_TILE, ROWS_PER_TILE)
    for k in range(ROWS_PER_TILE // CHUNK):
        rk = pl.multiple_of(r0 + k * CHUNK, CHUNK)
        pltpu.sync_copy(zf.at[pl.ds(rk, CHUNK)], rows0)
        pltpu.sync_copy(rows0, acc_sh.at[pl.ds(rk, CHUNK)])
    plsc.subcore_barrier()

    nchunk = jnp.where(cid == 0, AK0, AK1)
    base = pl.multiple_of(
        jnp.where(cid == 0, sid * AK0, 16 * AK0 + sid * AK1) * CHUNK, CHUNK)

    def idx_start(c, sb, db, semi):
        off = pl.multiple_of(base + c * CHUNK, CHUNK)
        pltpu.async_copy(srcp.at[pl.ds(off, CHUNK)], sb, semi)
        pltpu.async_copy(dstp.at[pl.ds(off, CHUNK)], db, semi)

    def idx_wait(c, sb, db, semi):
        off = pl.multiple_of(base + c * CHUNK, CHUNK)
        pltpu.make_async_copy(srcp.at[pl.ds(off, CHUNK)], sb, semi).wait()
        pltpu.make_async_copy(dstp.at[pl.ds(off, CHUNK)], db, semi).wait()

    idx_start(0, sidx0, didx0, semi0)
    idx_start(1, sidx1, didx1, semi1)
    idx_wait(0, sidx0, didx0, semi0)
    pltpu.async_copy(feat.at[sidx0], rows0, semg0)
    idx_wait(1, sidx1, didx1, semi1)

    def pair(i, carry):
        a = 2 * i
        pltpu.async_copy(feat.at[sidx1], rows1, semg1)
        pltpu.make_async_copy(feat.at[sidx0], rows0, semg0).wait()
        pltpu.sync_copy(rows0, acc_sh.at[didx0], add=True)
        idx_start(a + 2, sidx0, didx0, semi0)
        pltpu.make_async_copy(feat.at[sidx1], rows1, semg1).wait()
        pltpu.sync_copy(rows1, acc_sh.at[didx1], add=True)

        @pl.when(a + 3 < nchunk)
        def _():
            idx_start(a + 3, sidx1, didx1, semi1)

        idx_wait(a + 2, sidx0, didx0, semi0)

        @pl.when(a + 3 < nchunk)
        def _():
            idx_wait(a + 3, sidx1, didx1, semi1)

        pltpu.async_copy(feat.at[sidx0], rows0, semg0)
        return carry

    lax.fori_loop(0, (nchunk - 1) // 2, pair, 0)
    pltpu.make_async_copy(feat.at[sidx0], rows0, semg0).wait()
    pltpu.sync_copy(rows0, acc_sh.at[didx0], add=True)
    plsc.subcore_barrier()
    for k in range(ROWS_PER_TILE // CHUNK):
        rk = pl.multiple_of(r0 + k * CHUNK, CHUNK)
        pltpu.sync_copy(acc_sh.at[pl.ds(rk, CHUNK)], rows0)
        pltpu.sync_copy(rows0, part.at[cid, pl.ds(rk, CHUNK)])


def _agg_call(feat, srcp, dstp, zf):
    return pl.kernel(
        _agg_body,
        out_type=jax.ShapeDtypeStruct((2, NPAD, HID), jnp.float32),
        mesh=plsc.VectorSubcoreMesh(core_axis_name="c", subcore_axis_name="s",
                                    num_cores=2, num_subcores=16),
        scratch_types=[
            pltpu.VMEM((CHUNK,), jnp.int32),
            pltpu.VMEM((CHUNK,), jnp.int32),
            pltpu.VMEM((CHUNK,), jnp.int32),
            pltpu.VMEM((CHUNK,), jnp.int32),
            pltpu.VMEM((CHUNK, HID), jnp.float32),
            pltpu.VMEM((CHUNK, HID), jnp.float32),
            pltpu.SemaphoreType.DMA,
            pltpu.SemaphoreType.DMA,
            pltpu.SemaphoreType.DMA,
            pltpu.SemaphoreType.DMA,
            pltpu.VMEM_SHARED((NPAD, HID), jnp.float32),
        ],
    )(feat, srcp, dstp, zf)


def _cnt_body(dstp, zf, ones_h, cntp,
              didx0, didx1, ones_v, rows_v, semi0, semi1, acc_sh):
    cid = lax.axis_index("c")
    sid = lax.axis_index("s")
    wid = cid * 16 + sid
    r0 = pl.multiple_of(sid * ROWS_PER_TILE, ROWS_PER_TILE)
    for k in range(ROWS_PER_TILE // CHUNK):
        rk = pl.multiple_of(r0 + k * CHUNK, CHUNK)
        pltpu.sync_copy(zf.at[pl.ds(rk, CHUNK)], rows_v)
        pltpu.sync_copy(rows_v, acc_sh.at[pl.ds(rk, CHUNK)])
    pltpu.sync_copy(ones_h, ones_v)
    base = wid * EPT

    def idx_start(c, db, semi):
        off = pl.multiple_of(base + c * CHUNK, CHUNK)
        pltpu.async_copy(dstp.at[pl.ds(off, CHUNK)], db, semi)

    def idx_wait(c, db, semi):
        off = pl.multiple_of(base + c * CHUNK, CHUNK)
        pltpu.make_async_copy(dstp.at[pl.ds(off, CHUNK)], db, semi).wait()

    idx_start(0, didx0, semi0)
    idx_start(1, didx1, semi1)
    plsc.subcore_barrier()

    def pair(i, carry):
        a = 2 * i
        idx_wait(a, didx0, semi0)
        pltpu.sync_copy(ones_v, acc_sh.at[didx0], add=True)
        idx_start(a + 2, didx0, semi0)
        idx_wait(a + 1, didx1, semi1)
        pltpu.sync_copy(ones_v, acc_sh.at[didx1], add=True)

        @pl.when(a + 3 < NCHUNK)
        def _():
            idx_start(a + 3, didx1, semi1)

        return carry

    lax.fori_loop(0, (NCHUNK - 1) // 2, pair, 0)
    idx_wait(NCHUNK - 1, didx0, semi0)
    pltpu.sync_copy(ones_v, acc_sh.at[didx0], add=True)
    plsc.subcore_barrier()
    for k in range(ROWS_PER_TILE // CHUNK):
        rk = pl.multiple_of(r0 + k * CHUNK, CHUNK)
        pltpu.sync_copy(acc_sh.at[pl.ds(rk, CHUNK)], rows_v)
        pltpu.sync_copy(rows_v, cntp.at[cid, pl.ds(rk, CHUNK)])


def _cnt_call(dstp, zf, ones_h):
    return pl.kernel(
        _cnt_body,
        out_type=jax.ShapeDtypeStruct((2, NPAD, HID), jnp.float32),
        mesh=plsc.VectorSubcoreMesh(core_axis_name="c", subcore_axis_name="s",
                                    num_cores=2, num_subcores=16),
        scratch_types=[
            pltpu.VMEM((CHUNK,), jnp.int32),
            pltpu.VMEM((CHUNK,), jnp.int32),
            pltpu.VMEM((CHUNK, HID), jnp.float32),
            pltpu.VMEM((CHUNK, HID), jnp.float32),
            pltpu.SemaphoreType.DMA,
            pltpu.SemaphoreType.DMA,
            pltpu.VMEM_SHARED((NPAD, HID), jnp.float32),
        ],
    )(dstp, zf, ones_h)


def _sage1_body(p0, p1, c0, c1, x, wl, bl, wr, lnw, lnb, out):
    agg = (p0[...] + p1[...]) / jnp.maximum(c0[...] + c1[...], 1.0)
    y = (jnp.dot(agg, wl[...], preferred_element_type=jnp.float32)
         + bl[...]
         + jnp.dot(x[...], wr[...], preferred_element_type=jnp.float32))
    h1 = jnp.maximum(y, 0.0)
    mu = jnp.mean(h1, axis=1, keepdims=True)
    var = jnp.mean((h1 - mu) * (h1 - mu), axis=1, keepdims=True)
    x1 = (h1 - mu) * lax.rsqrt(var + 1e-5) * lnw[...] + lnb[...]
    out[...] = jnp.maximum(x1 + x[...], 0.0)


def _sage1_call(p0, p1, c0, c1, x, wlT, bl, wrT, lnw, lnb):
    spec_f = pl.BlockSpec((BLK, HID), lambda i: (i, 0))
    spec_w = pl.BlockSpec((HID, HID), lambda i: (0, 0))
    spec_b = pl.BlockSpec((1, HID), lambda i: (0, 0))
    return pl.pallas_call(
        _sage1_body,
        grid=(NPAD // BLK,),
        in_specs=[spec_f, spec_f, spec_f, spec_f, spec_f, spec_w, spec_b,
                  spec_w, spec_b, spec_b],
        out_specs=spec_f,
        out_shape=jax.ShapeDtypeStruct((NPAD, HID), jnp.float32),
    )(p0, p1, c0, c1, x, wlT, bl, wrT, lnw, lnb)


def _sage2_body(p0, p1, c0, c1, x1, res, wl, bl, wr, pw, pb, out):
    agg = (p0[...] + p1[...]) / jnp.maximum(c0[...] + c1[...], 1.0)
    y = (jnp.dot(agg, wl[...], preferred_element_type=jnp.float32)
         + bl[...]
         + jnp.dot(x1[...], wr[...], preferred_element_type=jnp.float32))
    xo = jnp.maximum(y + res[...], 0.0)
    o = jnp.dot(xo, pw[...], preferred_element_type=jnp.float32) + pb[...]
    rowid = (pl.program_id(0) * BLK
             + lax.broadcasted_iota(jnp.int32, (BLK, OUTP), 0))
    out[...] = jnp.where(rowid < N, o, 0.0)


def _sage2_call(p0, p1, c0, c1, x1, res, wl2T, bl2, wr2T, projWT, projb):
    spec_f = pl.BlockSpec((BLK, HID), lambda i: (i, 0))
    spec_w = pl.BlockSpec((HID, HID), lambda i: (0, 0))
    spec_b = pl.BlockSpec((1, HID), lambda i: (0, 0))
    return pl.pallas_call(
        _sage2_body,
        grid=(NPAD // BLK,),
        in_specs=[spec_f, spec_f, spec_f, spec_f, spec_f, spec_f, spec_w,
                  spec_b, spec_w,
                  pl.BlockSpec((HID, OUTP), lambda i: (0, 0)),
                  pl.BlockSpec((1, OUTP), lambda i: (0, 0))],
        out_specs=pl.BlockSpec((BLK, OUTP), lambda i: (i, 0)),
        out_shape=jax.ShapeDtypeStruct((NPAD, OUTP), jnp.float32),
    )(p0, p1, c0, c1, x1, res, wl2T, bl2, wr2T, projWT, projb)


KB_NCH = KB_PT // KB_CH


def _kb_sum(rows, out_v):
    for t in range(KB_CH):
        for half in range(2):
            acc = rows[t * BOM, pl.ds(half * 16, 16)]
            for j in range(1, BOM):
                acc = acc + rows[t * BOM + j, pl.ds(half * 16, 16)]
            out_v[t, pl.ds(half * 16, 16)] = acc


def _kb_body(proj, kbf, out, kidx, rows0, rows1, out_v, sem0, sem1):
    cid = lax.axis_index("c")
    sid = lax.axis_index("s")
    nch = jnp.where(cid == 0, KB0, KB1)
    tbase = pl.multiple_of(
        jnp.where(cid == 0, sid * KB0, 16 * KB0 + sid * KB1) * KB_CH, KB_CH)
    ibase = pl.multiple_of(tbase * BOM, KB_CH * BOM)
    pltpu.sync_copy(kbf.at[pl.ds(ibase, KBMX * KB_CH * BOM)], kidx)

    def emit(c, rows):
        _kb_sum(rows, out_v)
        toff = pl.multiple_of(tbase + c * KB_CH, KB_CH)
        pltpu.sync_copy(out_v, out.at[pl.ds(toff, KB_CH)])

    def gidx(c):
        return kidx.at[pl.ds(c * KB_CH * BOM, KB_CH * BOM)]

    pltpu.async_copy(proj.at[gidx(0)], rows0, sem0)

    def pair(i, carry):
        a = 2 * i
        pltpu.async_copy(proj.at[gidx(a + 1)], rows1, sem1)
        pltpu.make_async_copy(proj.at[gidx(a)], rows0, sem0).wait()
        emit(a, rows0)

        @pl.when(a + 2 < nch)
        def _():
            pltpu.async_copy(proj.at[gidx(a + 2)], rows0, sem0)

        pltpu.make_async_copy(proj.at[gidx(a + 1)], rows1, sem1).wait()
        emit(a + 1, rows1)
        return carry

    lax.fori_loop(0, nch // 2, pair, 0)


def _kb_call(proj, kbf):
    return pl.kernel(
        _kb_body,
        out_type=jax.ShapeDtypeStruct((NPAD, OUTP), jnp.float32),
        mesh=plsc.VectorSubcoreMesh(core_axis_name="c", subcore_axis_name="s",
                                    num_cores=2, num_subcores=16),
        scratch_types=[
            pltpu.VMEM((KBMX * KB_CH * BOM,), jnp.int32),
            pltpu.VMEM((KB_CH * BOM, OUTP), jnp.float32),
            pltpu.VMEM((KB_CH * BOM, OUTP), jnp.float32),
            pltpu.VMEM((KB_CH, OUTP), jnp.float32),
            pltpu.SemaphoreType.DMA,
            pltpu.SemaphoreType.DMA,
        ],
    )(proj, kbf)


def kernel(x_key, keybom, key_aggregation_status, edge_index,
           lstm_Wih, lstm_Whh, lstm_bih, lstm_bhh,
           sage1_Wl, sage1_bl, sage1_Wr, ln1_w, ln1_b,
           sage2_Wl, sage2_bl, sage2_Wr, proj_W, proj_b):
    f32 = jnp.float32
    x_t = jnp.zeros((TPAD, NPAD), f32).at[:T_IN, :N].set(x_key.T)
    wih_row = lstm_Wih.reshape(1, G4)
    whhT = lstm_Whh.T.astype(jnp.bfloat16)
    bias = (lstm_bih + lstm_bhh).reshape(1, G4)

    src = edge_index[0]
    dst = edge_index[1]
    srcp = jnp.full((EPAD,), N, jnp.int32).at[:E].set(src)
    dstp = jnp.full((EPAD,), N, jnp.int32).at[:E].set(dst)
    zf = jnp.zeros((NPAD, HID), f32)
    ones_h = jnp.ones((CHUNK, HID), f32)

    h = _lstm_call(x_t, wih_row, whhT, bias)
    res = h
    cnt = _cnt_call(dstp, zf, ones_h)

    part1 = _agg_call(h, srcp, dstp, zf)
    x1 = _sage1_call(part1[0], part1[1], cnt[0], cnt[1], h,
                     sage1_Wl.T, sage1_bl.reshape(1, HID), sage1_Wr.T,
                     ln1_w.reshape(1, HID), ln1_b.reshape(1, HID))

    part2 = _agg_call(x1, srcp, dstp, zf)
    projWT = jnp.zeros((HID, OUTP), f32).at[:, :OUT_CH].set(proj_W.T)
    projb = jnp.zeros((1, OUTP), f32).at[0, :OUT_CH].set(proj_b)
    po = _sage2_call(part2[0], part2[1], cnt[0], cnt[1], x1, res,
                     sage2_Wl.T, sage2_bl.reshape(1, HID), sage2_Wr.T,
                     projWT, projb)

    kb = jnp.where(keybom < 0, N, keybom)
    kbf = jnp.zeros((KBF_LEN,), jnp.int32).at[:N * BOM].set(kb.reshape(-1))
    ko = _kb_call(po, kbf)
    return ko[:N, :OUT_CH].reshape(N, TIME_STEPS, N_QUANTILES)
